# Initial kernel scaffold; baseline (speedup 1.0000x reference)
import jax
import jax.numpy as jnp
from jax.experimental import pallas as pl


def kernel(x_author, edge_index, paper_emb, Wp, bp, W1wl, b1w, W1wr, W1bl, b1b, W1br, W2wl, b2w, W2wr, W2bl, b2b, W2br, Wl1, bl1, Wl2, bl2):
    raise NotImplementedError("write your pallas kernel here")


# speedup vs baseline: 7.4092x; 7.4092x over previous
import functools

import jax
import jax.numpy as jnp
from jax import lax
from jax.experimental import pallas as pl
from jax.experimental.pallas import tpu as pltpu
from jax.experimental.pallas import tpu_sc as plsc

N = 50000
E = 800000
D_IN = 128
H = 64
H2 = 32
NP = N + 16
NTILES = 16
KCH = 4
KCH_C = 8
ROWS_E = 6272
EPAD = ROWS_E * 128
RPT = ROWS_E // NTILES
CHUNKS = RPT // KCH
CHUNKS_C = RPT // KCH_C
TPT = 3128
TPT_LAST = N - 15 * TPT
GROW, SROW = 0, 1


def _make_agg(grow, srow):
    @functools.partial(
        pl.kernel,
        mesh=plsc.VectorSubcoreMesh(core_axis_name="c", subcore_axis_name="s"),
        compiler_params=pltpu.CompilerParams(use_tc_tiling_on_sc=False),
        out_type=jax.ShapeDtypeStruct((2 * N, H2), jnp.float32),
        scratch_types=[
            pltpu.VMEM((KCH, 2, 128), jnp.int32),
            pltpu.VMEM((KCH * 128, H2), jnp.float32),
            pltpu.VMEM_SHARED((NP, H2), jnp.float32),
            pltpu.SemaphoreType.DMA,
            pltpu.SemaphoreType.DMA,
        ],
    )
    def agg(table, epack, zeros, out, ebuf, rows, acc, semg, sems):
        c = lax.axis_index("c")
        s = lax.axis_index("s")
        goff = c * NP
        noff = s * TPT

        @pl.when(s < NTILES - 1)
        def _():
            pltpu.sync_copy(zeros, acc.at[pl.ds(noff, TPT)])

        @pl.when(s == NTILES - 1)
        def _():
            pltpu.sync_copy(zeros.at[pl.ds(0, TPT_LAST)],
                            acc.at[pl.ds(noff, TPT_LAST)])

        plsc.subcore_barrier()

        def chunk(i, carry):
            rbase = s * RPT + i * KCH
            pltpu.sync_copy(epack.at[pl.ds(rbase, KCH)], ebuf)
            for j in range(KCH):
                def fixk(k, cr, j=j):
                    ebuf[j, grow, pl.ds(k * 16, 16)] = (
                        ebuf[j, grow, pl.ds(k * 16, 16)] + goff)
                    return cr
                lax.fori_loop(0, 8, fixk, 0)
            hs = [
                pltpu.async_copy(
                    table.at[ebuf.at[j, grow]],
                    rows.at[pl.ds(j * 128, 128)], semg)
                for j in range(KCH)
            ]
            for h in hs:
                h.wait()
            hs2 = [
                pltpu.async_copy(
                    rows.at[pl.ds(j * 128, 128)],
                    acc.at[ebuf.at[j, srow]], sems, add=True)
                for j in range(KCH)
            ]
            for h in hs2:
                h.wait()
            return carry

        lax.fori_loop(0, CHUNKS, chunk, 0)
        plsc.subcore_barrier()

        @pl.when(s < NTILES - 1)
        def _():
            pltpu.sync_copy(acc.at[pl.ds(noff, TPT)],
                            out.at[pl.ds(c * N + noff, TPT)])

        @pl.when(s == NTILES - 1)
        def _():
            pltpu.sync_copy(acc.at[pl.ds(noff, TPT_LAST)],
                            out.at[pl.ds(c * N + noff, TPT_LAST)])

    return agg


@functools.partial(
    pl.kernel,
    mesh=plsc.VectorSubcoreMesh(core_axis_name="c", subcore_axis_name="s"),
    compiler_params=pltpu.CompilerParams(use_tc_tiling_on_sc=False),
    out_type=jax.ShapeDtypeStruct((2 * N, 16), jnp.float32),
    scratch_types=[
        pltpu.VMEM((KCH_C, 2, 128), jnp.int32),
        pltpu.VMEM((128, 16), jnp.float32),
        pltpu.VMEM_SHARED((NP, 16), jnp.float32),
        pltpu.SemaphoreType.DMA,
    ],
)
def _counts(epack, ones_h, zeros, out, ebuf, ones_v, acc, sems):
    c = lax.axis_index("c")
    s = lax.axis_index("s")
    hrow = 1 - c
    noff = s * TPT

    @pl.when(s < NTILES - 1)
    def _():
        pltpu.sync_copy(zeros, acc.at[pl.ds(noff, TPT)])

    @pl.when(s == NTILES - 1)
    def _():
        pltpu.sync_copy(zeros.at[pl.ds(0, TPT_LAST)],
                        acc.at[pl.ds(noff, TPT_LAST)])

    pltpu.sync_copy(ones_h, ones_v)
    plsc.subcore_barrier()

    def chunk(i, carry):
        rbase = s * RPT + i * KCH_C
        pltpu.sync_copy(epack.at[pl.ds(rbase, KCH_C)], ebuf)
        hs = [
            pltpu.async_copy(ones_v, acc.at[ebuf.at[j, hrow]], sems, add=True)
            for j in range(KCH_C)
        ]
        for h in hs:
            h.wait()
        return carry

    lax.fori_loop(0, CHUNKS_C, chunk, 0)
    plsc.subcore_barrier()

    @pl.when(s < NTILES - 1)
    def _():
        pltpu.sync_copy(acc.at[pl.ds(noff, TPT)],
                        out.at[pl.ds(c * N + noff, TPT)])

    @pl.when(s == NTILES - 1)
    def _():
        pltpu.sync_copy(acc.at[pl.ds(noff, TPT_LAST)],
                        out.at[pl.ds(c * N + noff, TPT_LAST)])


def _proj_body(x, w, b, o):
    o[...] = jnp.dot(x[...], w[...], preferred_element_type=jnp.float32) + b[...]


def _proj(x, w, b):
    R = 1000
    return pl.pallas_call(
        _proj_body,
        grid=(N // R,),
        in_specs=[
            pl.BlockSpec((R, D_IN), lambda i: (i, 0)),
            pl.BlockSpec((D_IN, H), lambda i: (0, 0)),
            pl.BlockSpec((1, H), lambda i: (0, 0)),
        ],
        out_specs=pl.BlockSpec((R, H), lambda i: (i, 0)),
        out_shape=jax.ShapeDtypeStruct((N, H), jnp.float32),
    )(x, w, b)


def _combine_body(s0, s1, cnt, xd, wl, b, wr, o):
    m = jnp.concatenate([s0[...], s1[...]], axis=1)
    m = m / jnp.maximum(cnt[...][:, :1], 1.0)
    acc_v = jnp.dot(m, wl[...], preferred_element_type=jnp.float32)
    acc_v += jnp.dot(xd[...], wr[...], preferred_element_type=jnp.float32)
    o[...] = jnp.maximum(acc_v + b[...], 0.0)


def _combine(scat, cnt_cat, cnt_half, xd, wl, b, wr):
    R = 1000
    G = N // R
    return pl.pallas_call(
        _combine_body,
        grid=(G,),
        in_specs=[
            pl.BlockSpec((R, H2), lambda i: (i, 0)),
            pl.BlockSpec((R, H2), lambda i: (i + G, 0)),
            pl.BlockSpec((R, 16), lambda i: (i + cnt_half * G, 0)),
            pl.BlockSpec((R, H), lambda i: (i, 0)),
            pl.BlockSpec((H, H), lambda i: (0, 0)),
            pl.BlockSpec((1, H), lambda i: (0, 0)),
            pl.BlockSpec((H, H), lambda i: (0, 0)),
        ],
        out_specs=pl.BlockSpec((R, H), lambda i: (i, 0)),
        out_shape=jax.ShapeDtypeStruct((N, H), jnp.float32),
    )(scat, scat, cnt_cat, xd, wl, b, wr)


def _head_body(s0, s1, cnt, p1, wl, b, wr, w3, b3, w4, b4, o):
    m = jnp.concatenate([s0[...], s1[...]], axis=1)
    m = m / jnp.maximum(cnt[...][:, :1], 1.0)
    p2 = jnp.dot(m, wl[...], preferred_element_type=jnp.float32)
    p2 += jnp.dot(p1[...], wr[...], preferred_element_type=jnp.float32)
    p2 = jnp.maximum(p2 + b[...], 0.0)
    h = jnp.maximum(jnp.dot(p2, w3[...], preferred_element_type=jnp.float32) + b3[...], 0.0)
    o[...] = jnp.dot(h, w4[...], preferred_element_type=jnp.float32) + b4[...]


def _head(scat, cnt_cat, p1, wl, b, wr, w3, b3, w4, b4):
    R = 1000
    G = N // R
    return pl.pallas_call(
        _head_body,
        grid=(G,),
        in_specs=[
            pl.BlockSpec((R, H2), lambda i: (i, 0)),
            pl.BlockSpec((R, H2), lambda i: (i + G, 0)),
            pl.BlockSpec((R, 16), lambda i: (i, 0)),
            pl.BlockSpec((R, H), lambda i: (i, 0)),
            pl.BlockSpec((H, H), lambda i: (0, 0)),
            pl.BlockSpec((1, H), lambda i: (0, 0)),
            pl.BlockSpec((H, H), lambda i: (0, 0)),
            pl.BlockSpec((H, H), lambda i: (0, 0)),
            pl.BlockSpec((1, H), lambda i: (0, 0)),
            pl.BlockSpec((H, 1), lambda i: (0, 0)),
            pl.BlockSpec((1, 1), lambda i: (0, 0)),
        ],
        out_specs=pl.BlockSpec((R, 1), lambda i: (i, 0)),
        out_shape=jax.ShapeDtypeStruct((N, 1), jnp.float32),
    )(scat, scat, cnt_cat, p1, wl, b, wr, w3, b3, w4, b4)


def _split_cat(t):
    z = jnp.zeros((NP - N, H2), jnp.float32)
    return jnp.concatenate([t[:, :H2], z, t[:, H2:], z], axis=0)


_agg_fwd = _make_agg(GROW, SROW)
_agg_bwd = _make_agg(SROW, GROW)


def kernel(x_author, edge_index, paper_emb, Wp, bp,
           W1wl, b1w, W1wr, W1bl, b1b, W1br,
           W2wl, b2w, W2wr, W2bl, b2b, W2br,
           Wl1, bl1, Wl2, bl2):
    pad = jnp.full((2, EPAD - E), N, jnp.int32)
    epack = (jnp.concatenate([edge_index, pad], axis=1)
             .reshape(2, ROWS_E, 128).transpose(1, 0, 2))
    zeros32 = jnp.zeros((TPT, H2), jnp.float32)
    zeros16 = jnp.zeros((TPT, 16), jnp.float32)
    ones16 = jnp.ones((128, 16), jnp.float32)

    cnt = _counts(epack, ones16, zeros16)
    a = _proj(x_author, Wp, bp.reshape(1, H))
    pcat = _split_cat(paper_emb)
    Sa = _agg_bwd(pcat, epack, zeros32)
    acat = _split_cat(a)
    Sp = _agg_fwd(acat, epack, zeros32)
    p1 = _combine(Sp, cnt, 0, paper_emb, W1wl, b1w.reshape(1, H), W1wr)
    a1 = _combine(Sa, cnt, 1, a, W1bl, b1b.reshape(1, H), W1br)
    a1cat = _split_cat(a1)
    Sp2 = _agg_fwd(a1cat, epack, zeros32)
    return _head(Sp2, cnt, p1, W2wl, b2w.reshape(1, H), W2wr,
                 Wl1, bl1.reshape(1, H), Wl2, bl2.reshape(1, 1))
